# baseline (speedup 1.0000x reference)
import functools

import jax
import jax.numpy as jnp
from jax import lax
from jax.experimental import pallas as pl
from jax.experimental.pallas import tpu as pltpu
from jax.experimental.pallas import tpu_sc as plsc

F = 26
V = 100000
D = 16
B = 16384

NC = 2
NS = 16
NW = NC * NS

P = F * D
PPW = P // NW
L = 16

VA = 99968
TAIL = 32
HB = B // 2
U = 16


def _esmm_kernel(batch_t, tab_t, tail_t, out_t, plane_v, idx_v, out_v, tail_v,
                 semp, semo):
    wid = lax.axis_index("s") * NC + lax.axis_index("c")

    def fd(j):
        p = wid * PPW + j
        return p, p // D, p % D

    def fire_plane(j):
        _, f, d = fd(j)
        return pltpu.async_copy(tab_t.at[f, d, pl.ds(0, VA)],
                                plane_v.at[pl.ds(0, VA)], semp)

    def gather_half(h):
        @plsc.parallel_loop(0, HB, step=L, unroll=U)
        def _(o):
            iv = idx_v[pl.ds(o, L)]
            out_v[pl.ds(h * HB + o, L)] = plsc.load_gather(plane_v, [iv])

    cp = fire_plane(0)
    ocp = None
    for j in range(PPW):
        p, f, d = fd(j)
        pltpu.sync_copy(tail_t.at[f], tail_v)
        plane_v[pl.ds(V - 2 * L, L)] = tail_v[pl.ds(d * TAIL, L)]
        plane_v[pl.ds(V - L, L)] = tail_v[pl.ds(d * TAIL + L, L)]
        pltpu.sync_copy(batch_t.at[f, pl.ds(0, HB)], idx_v)
        cp.wait()
        if ocp is not None:
            ocp.wait()
        gather_half(0)
        pltpu.sync_copy(batch_t.at[f, pl.ds(HB, HB)], idx_v)
        gather_half(1)
        ocp = pltpu.async_copy(out_v, out_t.at[p], semo)
        if j + 1 < PPW:
            cp = fire_plane(j + 1)
    ocp.wait()


@jax.jit
def _esmm(batch, tables):
    batch_t = batch.astype(jnp.int32).T
    tab_t = tables.transpose(0, 2, 1)
    tail_t = lax.slice(tables, (0, V - TAIL, 0), (F, V, D)).transpose(
        0, 2, 1).reshape(F, D * TAIL)
    mesh = plsc.VectorSubcoreMesh(core_axis_name="c", subcore_axis_name="s")
    out_t = pl.kernel(
        _esmm_kernel,
        out_type=jax.ShapeDtypeStruct((P, B), jnp.float32),
        mesh=mesh,
        scratch_types=[
            pltpu.VMEM((V,), jnp.float32),
            pltpu.VMEM((HB,), jnp.int32),
            pltpu.VMEM((B,), jnp.float32),
            pltpu.VMEM((D * TAIL,), jnp.float32),
            pltpu.SemaphoreType.DMA,
            pltpu.SemaphoreType.DMA,
        ],
        compiler_params=pltpu.CompilerParams(
            use_tc_tiling_on_sc=True, needs_layout_passes=False),
    )(batch_t, tab_t, tail_t)
    return out_t.T.reshape(B, F * D)


def kernel(batch, tables):
    return _esmm(batch, tables)


# speedup vs baseline: 1.0528x; 1.0528x over previous
import functools

import jax
import jax.numpy as jnp
from jax import lax
from jax.experimental import pallas as pl
from jax.experimental.pallas import tpu as pltpu
from jax.experimental.pallas import tpu_sc as plsc

F = 26
V = 100000
D = 16
B = 16384

NC = 2
NS = 16
NW = NC * NS

P = F * D
PPW = P // NW
L = 16

VA = 99968
TAIL = 32
HB = B // 2
U = 8


def _esmm_kernel(batch_t, tab_t, tail_t, out_t, plane_v, idx_v, out_v, tail_v,
                 semp, semo):
    wid = lax.axis_index("s") * NC + lax.axis_index("c")

    def fd(j):
        p = wid * PPW + j
        return p, p // D, p % D

    def fire_plane(j):
        _, f, d = fd(j)
        return pltpu.async_copy(tab_t.at[f, d, pl.ds(0, VA)],
                                plane_v.at[pl.ds(0, VA)], semp)

    def gather_half(h):
        @plsc.parallel_loop(0, HB, step=L, unroll=U)
        def _(o):
            iv = idx_v[pl.ds(o, L)]
            out_v[pl.ds(h * HB + o, L)] = plsc.load_gather(plane_v, [iv])

    cp = fire_plane(0)
    ocp = None
    for j in range(PPW):
        p, f, d = fd(j)
        pltpu.sync_copy(tail_t.at[f], tail_v)
        plane_v[pl.ds(V - 2 * L, L)] = tail_v[pl.ds(d * TAIL, L)]
        plane_v[pl.ds(V - L, L)] = tail_v[pl.ds(d * TAIL + L, L)]
        h1st = j % 2
        h2nd = 1 - h1st
        if j == 0:
            pltpu.sync_copy(batch_t.at[f, pl.ds(h1st * HB, HB)], idx_v)
        else:
            _, f_prev, _ = fd(j - 1)

            @pl.when(f != f_prev)
            def _():
                pltpu.sync_copy(batch_t.at[f, pl.ds(h1st * HB, HB)], idx_v)

        cp.wait()
        if ocp is not None:
            ocp.wait()
        gather_half(h1st)
        pltpu.sync_copy(batch_t.at[f, pl.ds(h2nd * HB, HB)], idx_v)
        gather_half(h2nd)
        ocp = pltpu.async_copy(out_v, out_t.at[p], semo)
        if j + 1 < PPW:
            cp = fire_plane(j + 1)
    ocp.wait()


@jax.jit
def _esmm(batch, tables):
    batch_t = batch.astype(jnp.int32).T
    tab_t = tables.transpose(0, 2, 1)
    tail_t = lax.slice(tables, (0, V - TAIL, 0), (F, V, D)).transpose(
        0, 2, 1).reshape(F, D * TAIL)
    mesh = plsc.VectorSubcoreMesh(core_axis_name="c", subcore_axis_name="s")
    out_t = pl.kernel(
        _esmm_kernel,
        out_type=jax.ShapeDtypeStruct((P, B), jnp.float32),
        mesh=mesh,
        scratch_types=[
            pltpu.VMEM((V,), jnp.float32),
            pltpu.VMEM((HB,), jnp.int32),
            pltpu.VMEM((B,), jnp.float32),
            pltpu.VMEM((D * TAIL,), jnp.float32),
            pltpu.SemaphoreType.DMA,
            pltpu.SemaphoreType.DMA,
        ],
        compiler_params=pltpu.CompilerParams(
            use_tc_tiling_on_sc=True, needs_layout_passes=False),
    )(batch_t, tab_t, tail_t)
    return out_t.T.reshape(B, F * D)


def kernel(batch, tables):
    return _esmm(batch, tables)
